# Initial kernel scaffold; baseline (speedup 1.0000x reference)
import jax
import jax.numpy as jnp
from jax.experimental import pallas as pl


def kernel(entities, relations, x_coo, Wq, Wk, Wv, Wo, Wr, br, ln_g, ln_b, W_rs, b_rs):
    raise NotImplementedError("write your pallas kernel here")


# speedup vs baseline: 1.3202x; 1.3202x over previous
import functools

import jax
import jax.numpy as jnp
from jax import lax
from jax.experimental import pallas as pl
from jax.experimental.pallas import tpu as pltpu
from jax.experimental.pallas import tpu_sc as plsc

_N = 2048
_R = 8
_D = 128
_INV_SQRT_D = 1.0 / (128.0 ** 0.5)

_PREC = jax.lax.Precision.HIGHEST


def _dot(a, b, trans_b=False, precision=_PREC):
    dims = (((1,), (1 if trans_b else 0,)), ((), ()))
    return lax.dot_general(a, b, dims, precision=precision,
                           preferred_element_type=jnp.float32)


def _proj_body(ent_ref, rel_ref, wq_ref, wk_ref, wv_ref, wr_ref, br_ref,
               q_ref, k_ref, v_ref, rele_ref):
    ent = ent_ref[...]
    q_ref[...] = _dot(ent, wq_ref[...])
    k_ref[...] = _dot(ent, wk_ref[...])
    v_ref[...] = _dot(ent, wv_ref[...])
    rele_ref[...] = _dot(rel_ref[...], wr_ref[...]) + br_ref[...][None, :]


def _proj(entities, relations, Wq, Wk, Wv, Wr, br):
    return pl.pallas_call(
        _proj_body,
        out_shape=[
            jax.ShapeDtypeStruct((_N, _D), jnp.float32),
            jax.ShapeDtypeStruct((_N, _D), jnp.float32),
            jax.ShapeDtypeStruct((_N, _D), jnp.float32),
            jax.ShapeDtypeStruct((_R, _D), jnp.float32),
        ],
    )(entities, relations, Wq, Wk, Wv, Wr, br)


_ABLK = 256


def _att_body(q_ref, k_ref, rele_ref, ea_ref, eb_ref):
    q = q_ref[...]
    a = _dot(q, k_ref[...], trans_b=True) * _INV_SQRT_D
    ma = jnp.max(a, axis=1, keepdims=True)
    ea_ref[...] = jnp.exp(a - ma)
    b = _dot(q, rele_ref[...], trans_b=True) * _INV_SQRT_D
    mb = jnp.max(b, axis=1, keepdims=True)
    eb_ref[...] = jnp.exp(b - mb)


def _att(q, k, rel_e):
    nblk = _N // _ABLK
    return pl.pallas_call(
        _att_body,
        grid=(nblk,),
        in_specs=[
            pl.BlockSpec((_ABLK, _D), lambda i: (i, 0)),
            pl.BlockSpec((_N, _D), lambda i: (0, 0)),
            pl.BlockSpec((_R, _D), lambda i: (0, 0)),
        ],
        out_specs=[
            pl.BlockSpec((_ABLK, _N), lambda i: (i, 0)),
            pl.BlockSpec((_ABLK, _R), lambda i: (i, 0)),
        ],
        out_shape=[
            jax.ShapeDtypeStruct((_N, _N), jnp.float32),
            jax.ShapeDtypeStruct((_N, _R), jnp.float32),
        ],
    )(q, k, rel_e)


_MRBLK = 2048


def _mr_body(rele_ref, wrs_ref, brs_ref, mr_ref):
    mr_ref[...] = _dot(rele_ref[...], wrs_ref[...]) + brs_ref[...][None, :]


def _mr(rel_e, W_rs, b_rs):
    nblk = (_D * _D) // _MRBLK
    return pl.pallas_call(
        _mr_body,
        grid=(nblk,),
        in_specs=[
            pl.BlockSpec((_R, _D), lambda j: (0, 0)),
            pl.BlockSpec((_D, _MRBLK), lambda j: (0, j)),
            pl.BlockSpec((_MRBLK,), lambda j: (j,)),
        ],
        out_specs=pl.BlockSpec((_R, _MRBLK), lambda j: (0, j)),
        out_shape=jax.ShapeDtypeStruct((_R, _D * _D), jnp.float32),
    )(rel_e, W_rs, b_rs)


_GBLK = 256


def _agg_body(w_ref, ea_ref, u_ref, eb_ref, v_ref, rele_ref, ent_ref,
              wo_ref, g_ref, b_ref, out_ref):
    p = w_ref[...] * ea_ref[...]
    denom = jnp.sum(p, axis=1, keepdims=True) + 1e-30
    num = _dot(p, v_ref[...]) + _dot(u_ref[...] * eb_ref[...], rele_ref[...])
    agg = num / denom
    h = ent_ref[...] + _dot(agg, wo_ref[...])
    mu = jnp.mean(h, axis=1, keepdims=True)
    hc = h - mu
    var = jnp.mean(hc * hc, axis=1, keepdims=True)
    out_ref[...] = g_ref[...][None, :] * hc * jax.lax.rsqrt(var + 1e-5) \
        + b_ref[...][None, :]


def _agg(W, eA, U, eB, v, rel_e, entities, Wo, ln_g, ln_b):
    nblk = _N // _GBLK
    return pl.pallas_call(
        _agg_body,
        grid=(nblk,),
        in_specs=[
            pl.BlockSpec((_GBLK, _N), lambda i: (i, 0)),
            pl.BlockSpec((_GBLK, _N), lambda i: (i, 0)),
            pl.BlockSpec((_GBLK, _R), lambda i: (i, 0)),
            pl.BlockSpec((_GBLK, _R), lambda i: (i, 0)),
            pl.BlockSpec((_N, _D), lambda i: (0, 0)),
            pl.BlockSpec((_R, _D), lambda i: (0, 0)),
            pl.BlockSpec((_GBLK, _D), lambda i: (i, 0)),
            pl.BlockSpec((_D, _D), lambda i: (0, 0)),
            pl.BlockSpec((_D,), lambda i: (0,)),
            pl.BlockSpec((_D,), lambda i: (0,)),
        ],
        out_specs=pl.BlockSpec((_GBLK, _D), lambda i: (i, 0)),
        out_shape=jax.ShapeDtypeStruct((_N, _D), jnp.float32),
    )(W, eA, U, eB, v, rel_e, entities, Wo, ln_g, ln_b)


_SBLK = 256


def _rescal_body(mr_ref, emb_i_ref, emb_ref, out_ref):
    p = _PREC
    t = _dot(emb_i_ref[...], mr_ref[0], precision=p)
    out_ref[0] = _dot(t, emb_ref[...], trans_b=True, precision=p)


def _rescal(Mr3, ent_emb):
    nblk = _N // _SBLK
    return pl.pallas_call(
        _rescal_body,
        grid=(_R, nblk),
        in_specs=[
            pl.BlockSpec((1, _D, _D), lambda r, i: (r, 0, 0)),
            pl.BlockSpec((_SBLK, _D), lambda r, i: (i, 0)),
            pl.BlockSpec((_N, _D), lambda r, i: (0, 0)),
        ],
        out_specs=pl.BlockSpec((1, _SBLK, _N), lambda r, i: (r, i, 0)),
        out_shape=jax.ShapeDtypeStruct((_R, _N, _N), jnp.float32),
    )(Mr3, ent_emb, ent_emb)


def _edge_pass(x_coo, eA, eB):
    s = x_coo[:, 0]
    r = x_coo[:, 1]
    d = x_coo[:, 2]
    valB = eB[d, r]
    valA = eA[d, s]
    W = jnp.zeros((_N, _N), jnp.float32).at[d, s].add(valB)
    U = jnp.zeros((_N, _R), jnp.float32).at[d, r].add(valA)
    return W, U


def kernel(entities, relations, x_coo, Wq, Wk, Wv, Wo, Wr, br, ln_g, ln_b,
           W_rs, b_rs):
    q, k, v, rel_e = _proj(entities, relations, Wq, Wk, Wv, Wr, br)
    eA, eB = _att(q, k, rel_e)
    W, U = _edge_pass(x_coo, eA, eB)
    ent_emb = _agg(W, eA, U, eB, v, rel_e, entities, Wo, ln_g, ln_b)
    Mr = _mr(rel_e, W_rs, b_rs)
    Mr3 = Mr.reshape(_R, _D, _D)
    return _rescal(Mr3, ent_emb)


# speedup vs baseline: 5.5808x; 4.2273x over previous
import functools

import jax
import jax.numpy as jnp
from jax import lax
from jax.experimental import pallas as pl
from jax.experimental.pallas import tpu as pltpu
from jax.experimental.pallas import tpu_sc as plsc

_N = 2048
_R = 8
_D = 128
_INV_SQRT_D = 1.0 / (128.0 ** 0.5)

_PREC = jax.lax.Precision.HIGHEST


def _dot(a, b, trans_b=False, precision=_PREC):
    dims = (((1,), (1 if trans_b else 0,)), ((), ()))
    return lax.dot_general(a, b, dims, precision=precision,
                           preferred_element_type=jnp.float32)


def _proj_body(ent_ref, rel_ref, wq_ref, wk_ref, wv_ref, wr_ref, br_ref,
               q_ref, k_ref, v_ref, rele_ref):
    ent = ent_ref[...]
    q_ref[...] = _dot(ent, wq_ref[...])
    k_ref[...] = _dot(ent, wk_ref[...])
    v_ref[...] = _dot(ent, wv_ref[...])
    rele_ref[...] = _dot(rel_ref[...], wr_ref[...]) + br_ref[...][None, :]


def _proj(entities, relations, Wq, Wk, Wv, Wr, br):
    return pl.pallas_call(
        _proj_body,
        out_shape=[
            jax.ShapeDtypeStruct((_N, _D), jnp.float32),
            jax.ShapeDtypeStruct((_N, _D), jnp.float32),
            jax.ShapeDtypeStruct((_N, _D), jnp.float32),
            jax.ShapeDtypeStruct((_R, _D), jnp.float32),
        ],
    )(entities, relations, Wq, Wk, Wv, Wr, br)


_ABLK = 256


def _att_body(q_ref, k_ref, rele_ref, ea_ref, eb_ref):
    q = q_ref[...]
    a = _dot(q, k_ref[...], trans_b=True) * _INV_SQRT_D
    ma = jnp.max(a, axis=1, keepdims=True)
    ea_ref[...] = jnp.exp(a - ma)
    b = _dot(q, rele_ref[...], trans_b=True) * _INV_SQRT_D
    mb = jnp.max(b, axis=1, keepdims=True)
    eb_ref[...] = jnp.exp(b - mb)


def _att(q, k, rel_e):
    nblk = _N // _ABLK
    return pl.pallas_call(
        _att_body,
        grid=(nblk,),
        in_specs=[
            pl.BlockSpec((_ABLK, _D), lambda i: (i, 0)),
            pl.BlockSpec((_N, _D), lambda i: (0, 0)),
            pl.BlockSpec((_R, _D), lambda i: (0, 0)),
        ],
        out_specs=[
            pl.BlockSpec((_ABLK, _N), lambda i: (i, 0)),
            pl.BlockSpec((_ABLK, _R), lambda i: (i, 0)),
        ],
        out_shape=[
            jax.ShapeDtypeStruct((_N, _N), jnp.float32),
            jax.ShapeDtypeStruct((_N, _R), jnp.float32),
        ],
    )(q, k, rel_e)


_MRBLK = 2048


def _mr_body(rele_ref, wrs_ref, brs_ref, mr_ref):
    mr_ref[...] = _dot(rele_ref[...], wrs_ref[...]) + brs_ref[...][None, :]


def _mr(rel_e, W_rs, b_rs):
    nblk = (_D * _D) // _MRBLK
    return pl.pallas_call(
        _mr_body,
        grid=(nblk,),
        in_specs=[
            pl.BlockSpec((_R, _D), lambda j: (0, 0)),
            pl.BlockSpec((_D, _MRBLK), lambda j: (0, j)),
            pl.BlockSpec((_MRBLK,), lambda j: (j,)),
        ],
        out_specs=pl.BlockSpec((_R, _MRBLK), lambda j: (0, j)),
        out_shape=jax.ShapeDtypeStruct((_R, _D * _D), jnp.float32),
    )(rel_e, W_rs, b_rs)


_GBLK = 256


def _agg_body(w0_ref, w1_ref, ea_ref, u_ref, eb_ref, v_ref, rele_ref, ent_ref,
              wo_ref, g_ref, b_ref, out_ref):
    ea = ea_ref[...]
    pl_ = w0_ref[...] * ea[:, :_QC]
    pr_ = w1_ref[...] * ea[:, _QC:]
    denom = (jnp.sum(pl_, axis=1, keepdims=True)
             + jnp.sum(pr_, axis=1, keepdims=True) + 1e-30)
    num = (_dot(pl_, v_ref[...][:_QC]) + _dot(pr_, v_ref[...][_QC:])
           + _dot(u_ref[...] * eb_ref[...], rele_ref[...]))
    agg = num / denom
    h = ent_ref[...] + _dot(agg, wo_ref[...])
    mu = jnp.mean(h, axis=1, keepdims=True)
    hc = h - mu
    var = jnp.mean(hc * hc, axis=1, keepdims=True)
    out_ref[...] = g_ref[...][None, :] * hc * jax.lax.rsqrt(var + 1e-5) \
        + b_ref[...][None, :]


def _agg(w0, w1, eA, U, eB, v, rel_e, entities, Wo, ln_g, ln_b):
    nblk = _N // _GBLK
    return pl.pallas_call(
        _agg_body,
        grid=(nblk,),
        in_specs=[
            pl.BlockSpec((_GBLK, _QC), lambda i: (i, 0)),
            pl.BlockSpec((_GBLK, _QC), lambda i: (i, 0)),
            pl.BlockSpec((_GBLK, _N), lambda i: (i, 0)),
            pl.BlockSpec((_GBLK, _R), lambda i: (i, 0)),
            pl.BlockSpec((_GBLK, _R), lambda i: (i, 0)),
            pl.BlockSpec((_N, _D), lambda i: (0, 0)),
            pl.BlockSpec((_R, _D), lambda i: (0, 0)),
            pl.BlockSpec((_GBLK, _D), lambda i: (i, 0)),
            pl.BlockSpec((_D, _D), lambda i: (0, 0)),
            pl.BlockSpec((_D,), lambda i: (0,)),
            pl.BlockSpec((_D,), lambda i: (0,)),
        ],
        out_specs=pl.BlockSpec((_GBLK, _D), lambda i: (i, 0)),
        out_shape=jax.ShapeDtypeStruct((_N, _D), jnp.float32),
    )(w0, w1, eA, U, eB, v, rel_e, entities, Wo, ln_g, ln_b)


_SBLK = 256


def _rescal_body(mr_ref, emb_i_ref, emb_ref, out_ref):
    p = _PREC
    t = _dot(emb_i_ref[...], mr_ref[0], precision=p)
    out_ref[0] = _dot(t, emb_ref[...], trans_b=True, precision=p)


def _rescal(Mr3, ent_emb):
    nblk = _N // _SBLK
    return pl.pallas_call(
        _rescal_body,
        grid=(_R, nblk),
        in_specs=[
            pl.BlockSpec((1, _D, _D), lambda r, i: (r, 0, 0)),
            pl.BlockSpec((_SBLK, _D), lambda r, i: (i, 0)),
            pl.BlockSpec((_N, _D), lambda r, i: (0, 0)),
        ],
        out_specs=pl.BlockSpec((1, _SBLK, _N), lambda r, i: (r, i, 0)),
        out_shape=jax.ShapeDtypeStruct((_R, _N, _N), jnp.float32),
    )(Mr3, ent_emb, ent_emb)


_E = 131072
_NT = 16
_EPT = _E // _NT
_NH = 2
_EPH = _EPT // _NH
_NCH = _EPH // 128
_QR = 1024
_QC = 1024
_QW = _QR * _QC
_TRASH_W = _QW
_UW = _QR * _R
_TRASH_U = _UW
_ZB = 2048
_WPT = _QW // _NT


def _edge_sc(coo_flat, ea_flat, eb_flat):
    mesh = plsc.VectorSubcoreMesh(core_axis_name="c", subcore_axis_name="s")

    @functools.partial(
        pl.kernel,
        out_type=[
            jax.ShapeDtypeStruct((2, _N * _QC), jnp.float32),
            jax.ShapeDtypeStruct((_N * _R,), jnp.float32),
        ],
        mesh=mesh,
        compiler_params=pltpu.CompilerParams(needs_layout_passes=False),
        scratch_types=[
            pltpu.VMEM_SHARED((_QW + 8,), jnp.float32),
            pltpu.VMEM_SHARED((_UW + 8,), jnp.float32),
            pltpu.VMEM((_EPH * 3,), jnp.int32),
            pltpu.VMEM((_N * _R,), jnp.float32),
            pltpu.VMEM((_NCH, 128), jnp.int32),
            pltpu.VMEM((_NCH, 128), jnp.int32),
            pltpu.VMEM((_NCH, 128), jnp.int32),
            pltpu.VMEM((_NCH, 128), jnp.float32),
            pltpu.VMEM((_NCH, 128), jnp.float32),
            pltpu.VMEM((_ZB,), jnp.float32),
            pltpu.SemaphoreType.DMA,
        ],
    )
    def k(coo_hbm, ea_hbm, eb_hbm, w_hbm, u_hbm, wq_sh, u_sh, coo_v, eb_v,
          idxw_v, idxu_v, idxa_v, valb_v, vala_v, zbuf, sem):
        c = lax.axis_index("c")
        t = lax.axis_index("s")
        rb = c * _QR
        iota = lax.iota(jnp.int32, 16)

        pltpu.sync_copy(eb_hbm, eb_v)

        def zb_body(i, _):
            zbuf[pl.ds(i * 16, 16)] = jnp.zeros((16,), jnp.float32)
            return 0
        lax.fori_loop(0, _ZB // 16, zb_body, 0)

        @pl.when(t == 0)
        def _():
            for i in range(_UW // _ZB):
                pltpu.sync_copy(zbuf, u_sh.at[pl.ds(i * _ZB, _ZB)])

        for p in range(2):
            cb = p * _QC
            for i in range(_WPT // _ZB):
                pltpu.sync_copy(zbuf,
                                wq_sh.at[pl.ds(t * _WPT + i * _ZB, _ZB)])
            plsc.subcore_barrier()

            for h in range(_NH):
                pltpu.sync_copy(
                    coo_hbm.at[pl.ds((t * _EPT + h * _EPH) * 3, _EPH * 3)],
                    coo_v)

                def grp_body(g, _):
                    j = g // 8
                    lofs = (g % 8) * 16
                    ebase = g * 48 + iota * 3
                    es = plsc.load_gather(coo_v, [ebase])
                    er = plsc.load_gather(coo_v, [ebase + 1])
                    ed = plsc.load_gather(coo_v, [ebase + 2])
                    mask = ((ed >= rb) & (ed < rb + _QR)
                            & (es >= cb) & (es < cb + _QC))
                    idxw = jnp.where(mask, (ed - rb) * _QC + (es - cb),
                                     _TRASH_W)
                    idxu = jnp.where(mask, (ed - rb) * _R + er, _TRASH_U)
                    idxa = ed * _N + es
                    valb = plsc.load_gather(eb_v, [ed * _R + er])
                    idxw_v[j, pl.ds(lofs, 16)] = idxw
                    idxu_v[j, pl.ds(lofs, 16)] = idxu
                    idxa_v[j, pl.ds(lofs, 16)] = idxa
                    valb_v[j, pl.ds(lofs, 16)] = valb
                    return 0
                lax.fori_loop(0, _EPH // 16, grp_body, 0)

                gathers = [
                    pltpu.async_copy(ea_hbm.at[idxa_v.at[jj]], vala_v.at[jj],
                                     sem)
                    for jj in range(_NCH)
                ]
                for cp in gathers:
                    cp.wait()

                for jj in range(_NCH):
                    pltpu.sync_copy(valb_v.at[jj], wq_sh.at[idxw_v.at[jj]],
                                    add=True)
                    pltpu.sync_copy(vala_v.at[jj], u_sh.at[idxu_v.at[jj]],
                                    add=True)
            plsc.subcore_barrier()

            pltpu.sync_copy(
                wq_sh.at[pl.ds(t * _WPT, _WPT)],
                w_hbm.at[p, pl.ds((rb + t * (_QR // _NT)) * _QC, _WPT)])
            plsc.subcore_barrier()

        @pl.when(t == 0)
        def _():
            pltpu.sync_copy(u_sh.at[pl.ds(0, _UW)],
                            u_hbm.at[pl.ds(c * _UW, _UW)])

    return k(coo_flat, ea_flat, eb_flat)


def _edge_pass(x_coo, eA, eB):
    coo_flat = x_coo.astype(jnp.int32).reshape(-1)
    w2, u_flat = _edge_sc(coo_flat, eA.reshape(-1), eB.reshape(-1))
    w0 = w2[0].reshape(_N, _QC)
    w1 = w2[1].reshape(_N, _QC)
    return w0, w1, u_flat.reshape(_N, _R)


def kernel(entities, relations, x_coo, Wq, Wk, Wv, Wo, Wr, br, ln_g, ln_b,
           W_rs, b_rs):
    q, k, v, rel_e = _proj(entities, relations, Wq, Wk, Wv, Wr, br)
    eA, eB = _att(q, k, rel_e)
    w0, w1, U = _edge_pass(x_coo, eA, eB)
    ent_emb = _agg(w0, w1, eA, U, eB, v, rel_e, entities, Wo, ln_g, ln_b)
    Mr = _mr(rel_e, W_rs, b_rs)
    Mr3 = Mr.reshape(_R, _D, _D)
    return _rescal(Mr3, ent_emb)


# speedup vs baseline: 5.7315x; 1.0270x over previous
import functools

import jax
import jax.numpy as jnp
from jax import lax
from jax.experimental import pallas as pl
from jax.experimental.pallas import tpu as pltpu
from jax.experimental.pallas import tpu_sc as plsc

_N = 2048
_R = 8
_D = 128
_INV_SQRT_D = 1.0 / (128.0 ** 0.5)

_PREC = jax.lax.Precision.HIGHEST


def _dot(a, b, trans_b=False, precision=_PREC):
    dims = (((1,), (1 if trans_b else 0,)), ((), ()))
    return lax.dot_general(a, b, dims, precision=precision,
                           preferred_element_type=jnp.float32)


def _proj_body(ent_ref, rel_ref, wq_ref, wk_ref, wv_ref, wr_ref, br_ref,
               q_ref, k_ref, v_ref, rele_ref):
    ent = ent_ref[...]
    q_ref[...] = _dot(ent, wq_ref[...])
    k_ref[...] = _dot(ent, wk_ref[...])
    v_ref[...] = _dot(ent, wv_ref[...])
    rele_ref[...] = _dot(rel_ref[...], wr_ref[...]) + br_ref[...][None, :]


def _proj(entities, relations, Wq, Wk, Wv, Wr, br):
    return pl.pallas_call(
        _proj_body,
        out_shape=[
            jax.ShapeDtypeStruct((_N, _D), jnp.float32),
            jax.ShapeDtypeStruct((_N, _D), jnp.float32),
            jax.ShapeDtypeStruct((_N, _D), jnp.float32),
            jax.ShapeDtypeStruct((_R, _D), jnp.float32),
        ],
    )(entities, relations, Wq, Wk, Wv, Wr, br)


_ABLK = 256


def _att_body(q_ref, k_ref, rele_ref, ea_ref, eb_ref):
    q = q_ref[...]
    a = _dot(q, k_ref[...], trans_b=True) * _INV_SQRT_D
    ma = jnp.max(a, axis=1, keepdims=True)
    ea_ref[...] = jnp.exp(a - ma)
    b = _dot(q, rele_ref[...], trans_b=True) * _INV_SQRT_D
    mb = jnp.max(b, axis=1, keepdims=True)
    eb_ref[...] = jnp.exp(b - mb)


def _att(q, k, rel_e):
    nblk = _N // _ABLK
    return pl.pallas_call(
        _att_body,
        grid=(nblk,),
        in_specs=[
            pl.BlockSpec((_ABLK, _D), lambda i: (i, 0)),
            pl.BlockSpec((_N, _D), lambda i: (0, 0)),
            pl.BlockSpec((_R, _D), lambda i: (0, 0)),
        ],
        out_specs=[
            pl.BlockSpec((_ABLK, _N), lambda i: (i, 0)),
            pl.BlockSpec((_ABLK, _R), lambda i: (i, 0)),
        ],
        out_shape=[
            jax.ShapeDtypeStruct((_N, _N), jnp.float32),
            jax.ShapeDtypeStruct((_N, _R), jnp.float32),
        ],
    )(q, k, rel_e)


_MRBLK = 2048


def _mr_body(rele_ref, wrs_ref, brs_ref, mr_ref):
    mr_ref[...] = _dot(rele_ref[...], wrs_ref[...]) + brs_ref[...][None, :]


def _mr(rel_e, W_rs, b_rs):
    nblk = (_D * _D) // _MRBLK
    return pl.pallas_call(
        _mr_body,
        grid=(nblk,),
        in_specs=[
            pl.BlockSpec((_R, _D), lambda j: (0, 0)),
            pl.BlockSpec((_D, _MRBLK), lambda j: (0, j)),
            pl.BlockSpec((_MRBLK,), lambda j: (j,)),
        ],
        out_specs=pl.BlockSpec((_R, _MRBLK), lambda j: (0, j)),
        out_shape=jax.ShapeDtypeStruct((_R, _D * _D), jnp.float32),
    )(rel_e, W_rs, b_rs)


_GBLK = 256


def _agg_body(w0_ref, w1_ref, ea_ref, u_ref, eb_ref, v_ref, rele_ref, ent_ref,
              wo_ref, g_ref, b_ref, out_ref):
    ea = ea_ref[...]
    pl_ = w0_ref[...] * ea[:, :_QC]
    pr_ = w1_ref[...] * ea[:, _QC:]
    denom = (jnp.sum(pl_, axis=1, keepdims=True)
             + jnp.sum(pr_, axis=1, keepdims=True) + 1e-30)
    num = (_dot(pl_, v_ref[...][:_QC]) + _dot(pr_, v_ref[...][_QC:])
           + _dot(u_ref[...] * eb_ref[...], rele_ref[...]))
    agg = num / denom
    h = ent_ref[...] + _dot(agg, wo_ref[...])
    mu = jnp.mean(h, axis=1, keepdims=True)
    hc = h - mu
    var = jnp.mean(hc * hc, axis=1, keepdims=True)
    out_ref[...] = g_ref[...][None, :] * hc * jax.lax.rsqrt(var + 1e-5) \
        + b_ref[...][None, :]


def _agg(w0, w1, eA, U, eB, v, rel_e, entities, Wo, ln_g, ln_b):
    nblk = _N // _GBLK
    return pl.pallas_call(
        _agg_body,
        grid=(nblk,),
        in_specs=[
            pl.BlockSpec((_GBLK, _QC), lambda i: (i, 0)),
            pl.BlockSpec((_GBLK, _QC), lambda i: (i, 0)),
            pl.BlockSpec((_GBLK, _N), lambda i: (i, 0)),
            pl.BlockSpec((_GBLK, _R), lambda i: (i, 0)),
            pl.BlockSpec((_GBLK, _R), lambda i: (i, 0)),
            pl.BlockSpec((_N, _D), lambda i: (0, 0)),
            pl.BlockSpec((_R, _D), lambda i: (0, 0)),
            pl.BlockSpec((_GBLK, _D), lambda i: (i, 0)),
            pl.BlockSpec((_D, _D), lambda i: (0, 0)),
            pl.BlockSpec((_D,), lambda i: (0,)),
            pl.BlockSpec((_D,), lambda i: (0,)),
        ],
        out_specs=pl.BlockSpec((_GBLK, _D), lambda i: (i, 0)),
        out_shape=jax.ShapeDtypeStruct((_N, _D), jnp.float32),
    )(w0, w1, eA, U, eB, v, rel_e, entities, Wo, ln_g, ln_b)


_SBLK = 256


def _rescal_body(mr_ref, emb_i_ref, emb_ref, out_ref):
    p = _PREC
    t = _dot(emb_i_ref[...], mr_ref[0], precision=p)
    out_ref[0] = _dot(t, emb_ref[...], trans_b=True, precision=p)


def _rescal(Mr3, ent_emb):
    nblk = _N // _SBLK
    return pl.pallas_call(
        _rescal_body,
        grid=(_R, nblk),
        in_specs=[
            pl.BlockSpec((1, _D, _D), lambda r, i: (r, 0, 0)),
            pl.BlockSpec((_SBLK, _D), lambda r, i: (i, 0)),
            pl.BlockSpec((_N, _D), lambda r, i: (0, 0)),
        ],
        out_specs=pl.BlockSpec((1, _SBLK, _N), lambda r, i: (r, i, 0)),
        out_shape=jax.ShapeDtypeStruct((_R, _N, _N), jnp.float32),
    )(Mr3, ent_emb, ent_emb)


_E = 131072
_NT = 16
_EPT = _E // _NT
_NH = 2
_EPH = _EPT // _NH
_NCH = _EPH // 128
_QR = 1024
_QC = 1024
_QW = _QR * _QC
_TRASH_W = _QW
_UW = _QR * _R
_TRASH_U = _UW
_ZB = 8192
_WPT = _QW // _NT


def _edge_sc(coo_flat, ea_flat, eb_flat):
    mesh = plsc.VectorSubcoreMesh(core_axis_name="c", subcore_axis_name="s")

    @functools.partial(
        pl.kernel,
        out_type=[
            jax.ShapeDtypeStruct((2, _N * _QC), jnp.float32),
            jax.ShapeDtypeStruct((_N * _R,), jnp.float32),
        ],
        mesh=mesh,
        compiler_params=pltpu.CompilerParams(needs_layout_passes=False),
        scratch_types=[
            pltpu.VMEM_SHARED((_QW + 8,), jnp.float32),
            pltpu.VMEM_SHARED((_UW + 8,), jnp.float32),
            pltpu.VMEM((_EPH * 3,), jnp.int32),
            pltpu.VMEM((_N * _R,), jnp.float32),
            pltpu.VMEM((_EPH,), jnp.int32),
            pltpu.VMEM((_EPH,), jnp.int32),
            pltpu.VMEM((_EPH,), jnp.int32),
            pltpu.VMEM((_EPH,), jnp.float32),
            pltpu.VMEM((_EPH,), jnp.float32),
            pltpu.VMEM((_ZB,), jnp.float32),
            pltpu.SemaphoreType.DMA,
        ],
    )
    def k(coo_hbm, ea_hbm, eb_hbm, w_hbm, u_hbm, wq_sh, u_sh, coo_v, eb_v,
          idxw_v, idxu_v, idxa_v, valb_v, vala_v, zbuf, sem):
        c = lax.axis_index("c")
        t = lax.axis_index("s")
        rb = c * _QR
        iota = lax.iota(jnp.int32, 16)

        pltpu.sync_copy(eb_hbm, eb_v)

        def zb_body(i, _):
            zbuf[pl.ds(i * 16, 16)] = jnp.zeros((16,), jnp.float32)
            return 0
        lax.fori_loop(0, _ZB // 16, zb_body, 0)

        @pl.when(t == 0)
        def _():
            for i in range(_UW // _ZB):
                pltpu.sync_copy(zbuf, u_sh.at[pl.ds(i * _ZB, _ZB)])

        for p in range(2):
            cb = p * _QC
            for i in range(_WPT // _ZB):
                pltpu.sync_copy(zbuf,
                                wq_sh.at[pl.ds(t * _WPT + i * _ZB, _ZB)])
            plsc.subcore_barrier()

            for h in range(_NH):
                pltpu.sync_copy(
                    coo_hbm.at[pl.ds((t * _EPT + h * _EPH) * 3, _EPH * 3)],
                    coo_v)

                def grp_body(g, _):
                    lofs = g * 16
                    ebase = g * 48 + iota * 3
                    es = plsc.load_gather(coo_v, [ebase])
                    er = plsc.load_gather(coo_v, [ebase + 1])
                    ed = plsc.load_gather(coo_v, [ebase + 2])
                    rowm = (ed >= rb) & (ed < rb + _QR)
                    mask = rowm & (es >= cb) & (es < cb + _QC)
                    idxw = jnp.where(mask, (ed - rb) * _QC + (es - cb),
                                     _TRASH_W)
                    valb = plsc.load_gather(eb_v, [ed * _R + er])
                    idxw_v[pl.ds(lofs, 16)] = idxw
                    valb_v[pl.ds(lofs, 16)] = valb
                    if p == 0:
                        idxu = jnp.where(rowm, (ed - rb) * _R + er, _TRASH_U)
                        idxu_v[pl.ds(lofs, 16)] = idxu
                        idxa_v[pl.ds(lofs, 16)] = ed * _N + es
                    return 0
                lax.fori_loop(0, _EPH // 16, grp_body, 0)

                if p == 0:
                    pltpu.async_copy(ea_hbm.at[idxa_v], vala_v, sem).wait()
                    pltpu.sync_copy(vala_v, u_sh.at[idxu_v], add=True)
                pltpu.sync_copy(valb_v, wq_sh.at[idxw_v], add=True)
            plsc.subcore_barrier()

            pltpu.sync_copy(
                wq_sh.at[pl.ds(t * _WPT, _WPT)],
                w_hbm.at[p, pl.ds((rb + t * (_QR // _NT)) * _QC, _WPT)])
            plsc.subcore_barrier()

        @pl.when(t == 0)
        def _():
            pltpu.sync_copy(u_sh.at[pl.ds(0, _UW)],
                            u_hbm.at[pl.ds(c * _UW, _UW)])

    return k(coo_flat, ea_flat, eb_flat)


def _edge_pass(x_coo, eA, eB):
    coo_flat = x_coo.astype(jnp.int32).reshape(-1)
    w2, u_flat = _edge_sc(coo_flat, eA.reshape(-1), eB.reshape(-1))
    w0 = w2[0].reshape(_N, _QC)
    w1 = w2[1].reshape(_N, _QC)
    return w0, w1, u_flat.reshape(_N, _R)


def kernel(entities, relations, x_coo, Wq, Wk, Wv, Wo, Wr, br, ln_g, ln_b,
           W_rs, b_rs):
    q, k, v, rel_e = _proj(entities, relations, Wq, Wk, Wv, Wr, br)
    eA, eB = _att(q, k, rel_e)
    w0, w1, U = _edge_pass(x_coo, eA, eB)
    ent_emb = _agg(w0, w1, eA, U, eB, v, rel_e, entities, Wo, ln_g, ln_b)
    Mr = _mr(rel_e, W_rs, b_rs)
    Mr3 = Mr.reshape(_R, _D, _D)
    return _rescal(Mr3, ent_emb)


# speedup vs baseline: 6.3867x; 1.1143x over previous
import functools

import jax
import jax.numpy as jnp
from jax import lax
from jax.experimental import pallas as pl
from jax.experimental.pallas import tpu as pltpu
from jax.experimental.pallas import tpu_sc as plsc

_N = 2048
_R = 8
_D = 128
_INV_SQRT_D = 1.0 / (128.0 ** 0.5)

_PREC = jax.lax.Precision.HIGHEST


def _dot(a, b, trans_b=False, precision=_PREC):
    dims = (((1,), (1 if trans_b else 0,)), ((), ()))
    return lax.dot_general(a, b, dims, precision=precision,
                           preferred_element_type=jnp.float32)


def _proj_body(ent_ref, rel_ref, wq_ref, wk_ref, wv_ref, wr_ref, br_ref,
               q_ref, k_ref, v_ref, rele_ref):
    ent = ent_ref[...]
    q_ref[...] = _dot(ent, wq_ref[...])
    k_ref[...] = _dot(ent, wk_ref[...])
    v_ref[...] = _dot(ent, wv_ref[...])
    rele_ref[...] = _dot(rel_ref[...], wr_ref[...]) + br_ref[...][None, :]


def _proj(entities, relations, Wq, Wk, Wv, Wr, br):
    return pl.pallas_call(
        _proj_body,
        out_shape=[
            jax.ShapeDtypeStruct((_N, _D), jnp.float32),
            jax.ShapeDtypeStruct((_N, _D), jnp.float32),
            jax.ShapeDtypeStruct((_N, _D), jnp.float32),
            jax.ShapeDtypeStruct((_R, _D), jnp.float32),
        ],
    )(entities, relations, Wq, Wk, Wv, Wr, br)


_ABLK = 256


def _att_body(q_ref, k_ref, rele_ref, ea_ref, eb_ref):
    q = q_ref[...]
    a = _dot(q, k_ref[...], trans_b=True) * _INV_SQRT_D
    ma = jnp.max(a, axis=1, keepdims=True)
    ea_ref[...] = jnp.exp(a - ma)
    b = _dot(q, rele_ref[...], trans_b=True) * _INV_SQRT_D
    mb = jnp.max(b, axis=1, keepdims=True)
    eb_ref[...] = jnp.exp(b - mb)


def _att(q, k, rel_e):
    nblk = _N // _ABLK
    return pl.pallas_call(
        _att_body,
        grid=(nblk,),
        in_specs=[
            pl.BlockSpec((_ABLK, _D), lambda i: (i, 0)),
            pl.BlockSpec((_N, _D), lambda i: (0, 0)),
            pl.BlockSpec((_R, _D), lambda i: (0, 0)),
        ],
        out_specs=[
            pl.BlockSpec((_ABLK, _N), lambda i: (i, 0)),
            pl.BlockSpec((_ABLK, _R), lambda i: (i, 0)),
        ],
        out_shape=[
            jax.ShapeDtypeStruct((_N, _N), jnp.float32),
            jax.ShapeDtypeStruct((_N, _R), jnp.float32),
        ],
    )(q, k, rel_e)


_MRBLK = 2048


def _mr_body(rele_ref, wrs_ref, brs_ref, mr_ref):
    mr_ref[...] = _dot(rele_ref[...], wrs_ref[...]) + brs_ref[...][None, :]


def _mr(rel_e, W_rs, b_rs):
    nblk = (_D * _D) // _MRBLK
    return pl.pallas_call(
        _mr_body,
        grid=(nblk,),
        in_specs=[
            pl.BlockSpec((_R, _D), lambda j: (0, 0)),
            pl.BlockSpec((_D, _MRBLK), lambda j: (0, j)),
            pl.BlockSpec((_MRBLK,), lambda j: (j,)),
        ],
        out_specs=pl.BlockSpec((_R, _MRBLK), lambda j: (0, j)),
        out_shape=jax.ShapeDtypeStruct((_R, _D * _D), jnp.float32),
    )(rel_e, W_rs, b_rs)


_GBLK = 256


def _agg_body(w0_ref, w1_ref, ea_ref, u_ref, eb_ref, v_ref, rele_ref, ent_ref,
              wo_ref, g_ref, b_ref, out_ref):
    ea = ea_ref[...]
    pl_ = w0_ref[...] * ea[:, :_QC]
    pr_ = w1_ref[...] * ea[:, _QC:]
    denom = (jnp.sum(pl_, axis=1, keepdims=True)
             + jnp.sum(pr_, axis=1, keepdims=True) + 1e-30)
    num = (_dot(pl_, v_ref[...][:_QC]) + _dot(pr_, v_ref[...][_QC:])
           + _dot(u_ref[...] * eb_ref[...], rele_ref[...]))
    agg = num / denom
    h = ent_ref[...] + _dot(agg, wo_ref[...])
    mu = jnp.mean(h, axis=1, keepdims=True)
    hc = h - mu
    var = jnp.mean(hc * hc, axis=1, keepdims=True)
    out_ref[...] = g_ref[...][None, :] * hc * jax.lax.rsqrt(var + 1e-5) \
        + b_ref[...][None, :]


def _agg(w0, w1, eA, U, eB, v, rel_e, entities, Wo, ln_g, ln_b):
    nblk = _N // _GBLK
    return pl.pallas_call(
        _agg_body,
        grid=(nblk,),
        in_specs=[
            pl.BlockSpec((_GBLK, _QC), lambda i: (i, 0)),
            pl.BlockSpec((_GBLK, _QC), lambda i: (i, 0)),
            pl.BlockSpec((_GBLK, _N), lambda i: (i, 0)),
            pl.BlockSpec((_GBLK, _R), lambda i: (i, 0)),
            pl.BlockSpec((_GBLK, _R), lambda i: (i, 0)),
            pl.BlockSpec((_N, _D), lambda i: (0, 0)),
            pl.BlockSpec((_R, _D), lambda i: (0, 0)),
            pl.BlockSpec((_GBLK, _D), lambda i: (i, 0)),
            pl.BlockSpec((_D, _D), lambda i: (0, 0)),
            pl.BlockSpec((_D,), lambda i: (0,)),
            pl.BlockSpec((_D,), lambda i: (0,)),
        ],
        out_specs=pl.BlockSpec((_GBLK, _D), lambda i: (i, 0)),
        out_shape=jax.ShapeDtypeStruct((_N, _D), jnp.float32),
    )(w0, w1, eA, U, eB, v, rel_e, entities, Wo, ln_g, ln_b)


_SBLK = 256


def _rescal_body(mr_ref, emb_i_ref, emb_ref, out_ref):
    p = _PREC
    t = _dot(emb_i_ref[...], mr_ref[0], precision=p)
    out_ref[0] = _dot(t, emb_ref[...], trans_b=True, precision=p)


def _rescal(Mr3, ent_emb):
    nblk = _N // _SBLK
    return pl.pallas_call(
        _rescal_body,
        grid=(_R, nblk),
        in_specs=[
            pl.BlockSpec((1, _D, _D), lambda r, i: (r, 0, 0)),
            pl.BlockSpec((_SBLK, _D), lambda r, i: (i, 0)),
            pl.BlockSpec((_N, _D), lambda r, i: (0, 0)),
        ],
        out_specs=pl.BlockSpec((1, _SBLK, _N), lambda r, i: (r, i, 0)),
        out_shape=jax.ShapeDtypeStruct((_R, _N, _N), jnp.float32),
    )(Mr3, ent_emb, ent_emb)


_E = 131072
_NT = 16
_EPT = _E // _NT
_NH = 2
_EPH = _EPT // _NH
_NCH = _EPH // 128
_QR = 1024
_QC = 1024
_QW = _QR * _QC
_TRASH_W = _QW
_UW = _QR * _R
_TRASH_U = _UW
_ZB = 8192
_WPT = _QW // _NT


def _edge_sc(coo_flat, ea_flat, eb_flat):
    mesh = plsc.VectorSubcoreMesh(core_axis_name="c", subcore_axis_name="s")

    @functools.partial(
        pl.kernel,
        out_type=[
            jax.ShapeDtypeStruct((2, _N * _QC), jnp.float32),
            jax.ShapeDtypeStruct((_N * _R,), jnp.float32),
        ],
        mesh=mesh,
        compiler_params=pltpu.CompilerParams(needs_layout_passes=False),
        scratch_types=[
            pltpu.VMEM_SHARED((_QW + 8,), jnp.float32),
            pltpu.VMEM_SHARED((_UW + 8,), jnp.float32),
            pltpu.VMEM((_EPH,), jnp.int32),
            pltpu.VMEM((_EPH,), jnp.int32),
            pltpu.VMEM((_EPH,), jnp.int32),
            pltpu.VMEM((_N * _R,), jnp.float32),
            pltpu.VMEM((_EPH,), jnp.int32),
            pltpu.VMEM((_EPH,), jnp.int32),
            pltpu.VMEM((_EPH,), jnp.int32),
            pltpu.VMEM((_EPH,), jnp.float32),
            pltpu.VMEM((_EPH,), jnp.float32),
            pltpu.VMEM((_ZB,), jnp.float32),
            pltpu.SemaphoreType.DMA,
        ],
    )
    def k(coos_hbm, coor_hbm, cood_hbm, ea_hbm, eb_hbm, w_hbm, u_hbm,
          wq_sh, u_sh, coo_sv, coo_rv, coo_dv, eb_v,
          idxw_v, idxu_v, idxa_v, valb_v, vala_v, zbuf, sem):
        c = lax.axis_index("c")
        t = lax.axis_index("s")
        rb = c * _QR
        iota = lax.iota(jnp.int32, 16)

        pltpu.sync_copy(eb_hbm, eb_v)

        def zb_body(i, _):
            zbuf[pl.ds(i * 16, 16)] = jnp.zeros((16,), jnp.float32)
            return 0
        lax.fori_loop(0, _ZB // 16, zb_body, 0)

        @pl.when(t == 0)
        def _():
            for i in range(_UW // _ZB):
                pltpu.sync_copy(zbuf, u_sh.at[pl.ds(i * _ZB, _ZB)])

        for p in range(2):
            cb = p * _QC
            for i in range(_WPT // _ZB):
                pltpu.sync_copy(zbuf,
                                wq_sh.at[pl.ds(t * _WPT + i * _ZB, _ZB)])
            plsc.subcore_barrier()

            for h in range(_NH):
                ebeg = t * _EPT + h * _EPH
                pltpu.sync_copy(coos_hbm.at[pl.ds(ebeg, _EPH)], coo_sv)
                pltpu.sync_copy(coor_hbm.at[pl.ds(ebeg, _EPH)], coo_rv)
                pltpu.sync_copy(cood_hbm.at[pl.ds(ebeg, _EPH)], coo_dv)

                def grp_body(g, _):
                    lofs = g * 16
                    es = coo_sv[pl.ds(lofs, 16)]
                    er = coo_rv[pl.ds(lofs, 16)]
                    ed = coo_dv[pl.ds(lofs, 16)]
                    rowm = (ed >= rb) & (ed < rb + _QR)
                    mask = rowm & (es >= cb) & (es < cb + _QC)
                    idxw = jnp.where(mask, (ed - rb) * _QC + (es - cb),
                                     _TRASH_W)
                    valb = plsc.load_gather(eb_v, [ed * _R + er])
                    idxw_v[pl.ds(lofs, 16)] = idxw
                    valb_v[pl.ds(lofs, 16)] = valb
                    if p == 0:
                        idxu = jnp.where(rowm, (ed - rb) * _R + er, _TRASH_U)
                        idxu_v[pl.ds(lofs, 16)] = idxu
                        idxa_v[pl.ds(lofs, 16)] = ed * _N + es
                    return 0
                lax.fori_loop(0, _EPH // 16, grp_body, 0)

                if p == 0:
                    pltpu.async_copy(ea_hbm.at[idxa_v], vala_v, sem).wait()
                    pltpu.sync_copy(vala_v, u_sh.at[idxu_v], add=True)
                pltpu.sync_copy(valb_v, wq_sh.at[idxw_v], add=True)
            plsc.subcore_barrier()

            pltpu.sync_copy(
                wq_sh.at[pl.ds(t * _WPT, _WPT)],
                w_hbm.at[p, pl.ds((rb + t * (_QR // _NT)) * _QC, _WPT)])
            plsc.subcore_barrier()

        @pl.when(t == 0)
        def _():
            pltpu.sync_copy(u_sh.at[pl.ds(0, _UW)],
                            u_hbm.at[pl.ds(c * _UW, _UW)])

    return k(coo_flat[0], coo_flat[1], coo_flat[2], ea_flat, eb_flat)


def _edge_pass(x_coo, eA, eB):
    coo_cols = x_coo.astype(jnp.int32).T.copy()
    w2, u_flat = _edge_sc(coo_cols, eA.reshape(-1), eB.reshape(-1))
    w0 = w2[0].reshape(_N, _QC)
    w1 = w2[1].reshape(_N, _QC)
    return w0, w1, u_flat.reshape(_N, _R)


def kernel(entities, relations, x_coo, Wq, Wk, Wv, Wo, Wr, br, ln_g, ln_b,
           W_rs, b_rs):
    q, k, v, rel_e = _proj(entities, relations, Wq, Wk, Wv, Wr, br)
    eA, eB = _att(q, k, rel_e)
    w0, w1, U = _edge_pass(x_coo, eA, eB)
    ent_emb = _agg(w0, w1, eA, U, eB, v, rel_e, entities, Wo, ln_g, ln_b)
    Mr = _mr(rel_e, W_rs, b_rs)
    Mr3 = Mr.reshape(_R, _D, _D)
    return _rescal(Mr3, ent_emb)
